# Initial kernel scaffold; baseline (speedup 1.0000x reference)
import jax
import jax.numpy as jnp
from jax.experimental import pallas as pl


def kernel(x, W_gate, W_up, W_down):
    raise NotImplementedError("write your pallas kernel here")


# speedup vs baseline: 1.1507x; 1.1507x over previous
import jax
import jax.numpy as jnp
from jax.experimental import pallas as pl

D_MODEL = 2048
D_FF = 768
BLK_T = 512


def _mlp_block(x_ref, wg_ref, wu_ref, wd_ref, o_ref):
    xb = x_ref[...].astype(jnp.bfloat16)
    gate = jnp.dot(xb, wg_ref[...], preferred_element_type=jnp.float32)
    up = jnp.dot(xb, wu_ref[...], preferred_element_type=jnp.float32)
    hidden = (jax.nn.silu(gate) * up).astype(jnp.bfloat16)
    o_ref[...] = jnp.dot(hidden, wd_ref[...], preferred_element_type=jnp.float32)


def kernel(x, W_gate, W_up, W_down):
    n_tokens, d_model = x.shape
    d_ff = W_gate.shape[1]
    wg = W_gate.astype(jnp.bfloat16)
    wu = W_up.astype(jnp.bfloat16)
    wd = W_down.astype(jnp.bfloat16)
    grid = (n_tokens // BLK_T,)
    return pl.pallas_call(
        _mlp_block,
        grid=grid,
        in_specs=[
            pl.BlockSpec((BLK_T, d_model), lambda i: (i, 0)),
            pl.BlockSpec((d_model, d_ff), lambda i: (0, 0)),
            pl.BlockSpec((d_model, d_ff), lambda i: (0, 0)),
            pl.BlockSpec((d_ff, d_model), lambda i: (0, 0)),
        ],
        out_specs=pl.BlockSpec((BLK_T, d_model), lambda i: (i, 0)),
        out_shape=jax.ShapeDtypeStruct((n_tokens, d_model), jnp.float32),
    )(x, wg, wu, wd)
